# Initial kernel scaffold; baseline (speedup 1.0000x reference)
import jax
import jax.numpy as jnp
from jax.experimental import pallas as pl


def kernel(x, W1, b1, W2, b2, ln_scale):
    raise NotImplementedError("write your pallas kernel here")


# speedup vs baseline: 1.8690x; 1.8690x over previous
import jax
import jax.numpy as jnp
from jax.experimental import pallas as pl


def _fused_mlp_kernel(x_ref, w1_ref, b1_ref, w2_ref, b2_ref, s_ref, o_ref):
    x = x_ref[...]
    h = jnp.dot(x, w1_ref[...], preferred_element_type=jnp.float32)
    h = jnp.maximum(h + b1_ref[...], 0.0)
    y = jnp.dot(h, w2_ref[...], preferred_element_type=jnp.float32)
    y = y + b2_ref[...] + x
    var = jnp.mean(y * y, axis=-1, keepdims=True)
    o_ref[...] = (y * jax.lax.rsqrt(var + 1e-6)) * s_ref[...]


def kernel(x, W1, b1, W2, b2, ln_scale):
    B, S, D = x.shape
    K, V = W1.shape
    M = B * S
    TM = 512

    xf = x.reshape(M, D)
    b1r = b1.reshape(1, V)
    b2r = b2.reshape(1, D)
    snr = ln_scale.reshape(1, D)

    out = pl.pallas_call(
        _fused_mlp_kernel,
        grid=(M // TM,),
        in_specs=[
            pl.BlockSpec((TM, D), lambda m: (m, 0)),
            pl.BlockSpec((K, V), lambda m: (0, 0)),
            pl.BlockSpec((1, V), lambda m: (0, 0)),
            pl.BlockSpec((V, D), lambda m: (0, 0)),
            pl.BlockSpec((1, D), lambda m: (0, 0)),
            pl.BlockSpec((1, D), lambda m: (0, 0)),
        ],
        out_specs=pl.BlockSpec((TM, D), lambda m: (m, 0)),
        out_shape=jax.ShapeDtypeStruct((M, D), jnp.float32),
    )(xf, W1, b1r, W2, b2r, snr)
    return out.reshape(B, S, D)
